# baseline (speedup 1.0000x reference)
import functools

import jax
import jax.numpy as jnp
from jax import lax
from jax.experimental import pallas as pl
from jax.experimental.pallas import tpu as pltpu
from jax.experimental.pallas import tpu_sc as plsc

BATCH = 16
SEQ = 2048
FEAT = 256
TOTAL = BATCH * SEQ * FEAT
NNZ = 1000000

NW = 32
WR = TOTAL // NW
ZBUF = 32768
W = 2048
ROWS = W // 128
LANES = 16
NP = 490 * W


def _scatter_body(flat_hbm, vals_hbm, woffs_hbm, out_hbm,
                  zbuf, idxbuf, valbuf, sidx, offrow):
    cid = lax.axis_index("c")
    sid = lax.axis_index("s")
    wid = sid * 2 + cid
    wbase = wid * WR
    wend = wbase + WR

    pltpu.sync_copy(woffs_hbm.at[wid], offrow)
    ov = offrow[...]
    lo = ov[0]
    hi = ov[1]

    zeros16 = jnp.zeros((LANES,), jnp.float32)

    def zb(t, c):
        zbuf[pl.ds(t * LANES, LANES)] = zeros16
        return c
    lax.fori_loop(0, ZBUF // LANES, zb, 0, unroll=8)

    for b in range(WR // ZBUF):
        pltpu.sync_copy(
            zbuf,
            out_hbm.at[pl.ds(pl.multiple_of(wbase + b * ZBUF, ZBUF), ZBUF)])

    lo_al = jnp.bitwise_and(lo, -W)
    nwin = (hi - lo_al + (W - 1)) // W

    def wbody(w, c):
        start = pl.multiple_of(lo_al + w * W, W)
        pltpu.sync_copy(flat_hbm.at[pl.ds(start, W)], idxbuf)
        pltpu.sync_copy(vals_hbm.at[pl.ds(start, W)], valbuf)
        for t in range(W // LANES):
            fv = idxbuf[pl.ds(t * LANES, LANES)]
            keep = (fv >= wbase) & (fv < wend)
            sidx[pl.ds(t * LANES, LANES)] = jnp.where(keep, fv, TOTAL)
        pltpu.sync_copy(valbuf, out_hbm.at[sidx])
        return c
    lax.fori_loop(0, nwin, wbody, 0)


@jax.jit
def _to_dense(flat2, vals2, woffs):
    mesh = plsc.VectorSubcoreMesh(core_axis_name="c", subcore_axis_name="s")
    f = functools.partial(
        pl.kernel,
        mesh=mesh,
        out_type=jax.ShapeDtypeStruct((TOTAL + 128,), jnp.float32),
        scratch_types=[
            pltpu.VMEM((ZBUF,), jnp.float32),
            pltpu.VMEM((W,), jnp.int32),
            pltpu.VMEM((W,), jnp.float32),
            pltpu.VMEM((W,), jnp.int32),
            pltpu.VMEM((LANES,), jnp.int32),
        ],
    )(_scatter_body)
    return f(flat2, vals2, woffs)


def kernel(indices, values):
    idx32 = indices.astype(jnp.int32)
    flat = (idx32[:, 0] * SEQ + idx32[:, 1]) * FEAT + idx32[:, 2]
    bounds = jnp.arange(NW + 1, dtype=jnp.int32) * WR
    offs = jnp.searchsorted(flat, bounds, side="left").astype(jnp.int32)
    woffs = jnp.zeros((NW, LANES), jnp.int32)
    woffs = woffs.at[:, 0].set(offs[:-1]).at[:, 1].set(offs[1:])
    flat_p = jnp.concatenate([flat, jnp.full((NP - NNZ,), TOTAL, jnp.int32)])
    vals_p = jnp.concatenate([values, jnp.zeros((NP - NNZ,), jnp.float32)])
    out = _to_dense(flat_p, vals_p, woffs)
    return out[:TOTAL].reshape(BATCH, SEQ, FEAT)
